# Initial kernel scaffold; baseline (speedup 1.0000x reference)
import functools

import jax
import jax.numpy as jnp
from jax import lax
from jax.experimental import pallas as pl
from jax.experimental.pallas import tpu as pltpu
from jax.experimental.pallas import tpu_sc as plsc

BATCH = 16384
HIST = 50
EMBED = 32
B = BATCH * HIST
NUM_CORES = 2
NUM_SUBCORES = 16
NW = NUM_CORES * NUM_SUBCORES
B_PER_W = B // NW
CHUNK = 1024
NCHUNK = B_PER_W // CHUNK


def _sc_gather(table, idx_flat):
    mesh = plsc.VectorSubcoreMesh(core_axis_name="c", subcore_axis_name="s")

    @functools.partial(
        pl.kernel,
        out_type=jax.ShapeDtypeStruct((B, EMBED), jnp.float32),
        mesh=mesh,
        scratch_types=[
            pltpu.VMEM((CHUNK,), jnp.int32),
            pltpu.VMEM((CHUNK, EMBED), jnp.float32),
            pltpu.SemaphoreType.DMA,
        ],
    )
    def k(table_hbm, idx_hbm, out_hbm, idx_v, rows_v, sem):
        wid = lax.axis_index("s") * NUM_CORES + lax.axis_index("c")
        base = wid * B_PER_W

        @pl.loop(0, NCHUNK)
        def _step(i):
            off = base + i * CHUNK
            pltpu.sync_copy(idx_hbm.at[pl.ds(off, CHUNK)], idx_v)
            pltpu.async_copy(table_hbm.at[idx_v], rows_v, sem).wait()
            pltpu.sync_copy(rows_v, out_hbm.at[pl.ds(off, CHUNK)])

    return k(table, idx_flat)


@jax.jit
def kernel(inputs, embedding):
    idx_flat = jnp.reshape(inputs, (B,)).astype(jnp.int32)
    out = _sc_gather(embedding, idx_flat)
    return jnp.reshape(out, (BATCH, HIST, EMBED))


# speedup vs baseline: 1.0951x; 1.0951x over previous
import functools

import jax
import jax.numpy as jnp
from jax import lax
from jax.experimental import pallas as pl
from jax.experimental.pallas import tpu as pltpu
from jax.experimental.pallas import tpu_sc as plsc

BATCH = 16384
HIST = 50
EMBED = 32
B = BATCH * HIST
NUM_CORES = 2
NUM_SUBCORES = 16
NW = NUM_CORES * NUM_SUBCORES
B_PER_W = B // NW
CHUNK = 1024
NCHUNK = B_PER_W // CHUNK


def _sc_gather(table, idx_flat):
    mesh = plsc.VectorSubcoreMesh(core_axis_name="c", subcore_axis_name="s")

    @functools.partial(
        pl.kernel,
        out_type=jax.ShapeDtypeStruct((B, EMBED), jnp.float32),
        mesh=mesh,
        scratch_types=[
            pltpu.VMEM((CHUNK,), jnp.int32),
            pltpu.VMEM((CHUNK, EMBED), jnp.float32),
            pltpu.SemaphoreType.DMA,
        ],
        compiler_params=pltpu.CompilerParams(use_tc_tiling_on_sc=False),
    )
    def k(table_hbm, idx_hbm, out_hbm, idx_v, rows_v, sem):
        wid = lax.axis_index("s") * NUM_CORES + lax.axis_index("c")
        base = wid * B_PER_W

        @pl.loop(0, NCHUNK)
        def _step(i):
            off = base + i * CHUNK
            pltpu.sync_copy(idx_hbm.at[pl.ds(off, CHUNK)], idx_v)
            pltpu.async_copy(table_hbm.at[idx_v], rows_v, sem).wait()
            pltpu.sync_copy(rows_v, out_hbm.at[pl.ds(off, CHUNK)])

    return k(table, idx_flat)


@jax.jit
def kernel(inputs, embedding):
    idx_flat = jnp.reshape(inputs, (B,)).astype(jnp.int32)
    out = _sc_gather(embedding, idx_flat)
    return jnp.reshape(out, (BATCH, HIST, EMBED))


# speedup vs baseline: 1.1117x; 1.0152x over previous
import functools

import jax
import jax.numpy as jnp
from jax import lax
from jax.experimental import pallas as pl
from jax.experimental.pallas import tpu as pltpu
from jax.experimental.pallas import tpu_sc as plsc

BATCH = 16384
HIST = 50
EMBED = 32
B = BATCH * HIST
NUM_CORES = 2
NUM_SUBCORES = 16
NW = NUM_CORES * NUM_SUBCORES
B_PER_W = B // NW
CHUNK = 1280
NCHUNK = B_PER_W // CHUNK


def _sc_gather(table, idx_flat):
    mesh = plsc.VectorSubcoreMesh(core_axis_name="c", subcore_axis_name="s")

    @functools.partial(
        pl.kernel,
        out_type=jax.ShapeDtypeStruct((B, EMBED), jnp.float32),
        mesh=mesh,
        scratch_types=[
            pltpu.VMEM((CHUNK,), jnp.int32),
            pltpu.VMEM((CHUNK,), jnp.int32),
            pltpu.VMEM((CHUNK, EMBED), jnp.float32),
            pltpu.VMEM((CHUNK, EMBED), jnp.float32),
            pltpu.SemaphoreType.DMA,
            pltpu.SemaphoreType.DMA,
            pltpu.SemaphoreType.DMA,
            pltpu.SemaphoreType.DMA,
            pltpu.SemaphoreType.DMA,
            pltpu.SemaphoreType.DMA,
        ],
        compiler_params=pltpu.CompilerParams(use_tc_tiling_on_sc=False),
    )
    def k(table_hbm, idx_hbm, out_hbm, idx0, idx1, rows0, rows1,
          i0, i1, g0, g1, s0, s1):
        wid = lax.axis_index("s") * NUM_CORES + lax.axis_index("c")
        base = wid * B_PER_W
        idx_v = (idx0, idx1)
        rows_v = (rows0, rows1)
        isem = (i0, i1)
        gsem = (g0, g1)
        ssem = (s0, s1)

        def idx_copy(i, b):
            return pltpu.make_async_copy(
                idx_hbm.at[pl.ds(base + i * CHUNK, CHUNK)], idx_v[b], isem[b])

        def gather_copy(i, b):
            return pltpu.make_async_copy(
                table_hbm.at[idx_v[b]], rows_v[b], gsem[b])

        def store_copy(j, b):
            return pltpu.make_async_copy(
                rows_v[b], out_hbm.at[pl.ds(base + j * CHUNK, CHUNK)],
                ssem[b])

        idx_copy(0, 0).start()
        idx_copy(1, 1).start()

        for i in range(NCHUNK + 1):
            b = i % 2
            if i < NCHUNK:
                idx_copy(i, b).wait()
                if i >= 2:
                    store_copy(i - 2, b).wait()
                gather_copy(i, b).start()
            if i >= 1:
                j = i - 1
                bj = j % 2
                gather_copy(j, bj).wait()
                store_copy(j, bj).start()
                if j + 2 < NCHUNK:
                    idx_copy(j + 2, bj).start()
        store_copy(NCHUNK - 2, NCHUNK % 2).wait()
        store_copy(NCHUNK - 1, (NCHUNK - 1) % 2).wait()

    return k(table, idx_flat)


@jax.jit
def kernel(inputs, embedding):
    idx_flat = jnp.reshape(inputs, (B,)).astype(jnp.int32)
    out = _sc_gather(embedding, idx_flat)
    return jnp.reshape(out, (BATCH, HIST, EMBED))


# speedup vs baseline: 1.1121x; 1.0003x over previous
import functools

import jax
import jax.numpy as jnp
from jax import lax
from jax.experimental import pallas as pl
from jax.experimental.pallas import tpu as pltpu
from jax.experimental.pallas import tpu_sc as plsc

BATCH = 16384
HIST = 50
EMBED = 32
B = BATCH * HIST
NUM_CORES = 2
NUM_SUBCORES = 16
NW = NUM_CORES * NUM_SUBCORES
B_PER_W = B // NW
CHUNK = 800
NCHUNK = B_PER_W // CHUNK
DEPTH = 4


def _sc_gather(table, idx_flat):
    mesh = plsc.VectorSubcoreMesh(core_axis_name="c", subcore_axis_name="s")

    @functools.partial(
        pl.kernel,
        out_type=jax.ShapeDtypeStruct((B, EMBED), jnp.float32),
        mesh=mesh,
        scratch_types=(
            [pltpu.VMEM((CHUNK,), jnp.int32) for _ in range(DEPTH)]
            + [pltpu.VMEM((CHUNK, EMBED), jnp.float32) for _ in range(DEPTH)]
            + [pltpu.SemaphoreType.DMA for _ in range(3 * DEPTH)]
        ),
        compiler_params=pltpu.CompilerParams(use_tc_tiling_on_sc=False),
    )
    def k(table_hbm, idx_hbm, out_hbm, *scratch):
        idx_v = scratch[:DEPTH]
        rows_v = scratch[DEPTH:2 * DEPTH]
        isem = scratch[2 * DEPTH:3 * DEPTH]
        gsem = scratch[3 * DEPTH:4 * DEPTH]
        ssem = scratch[4 * DEPTH:5 * DEPTH]
        wid = lax.axis_index("s") * NUM_CORES + lax.axis_index("c")
        base = wid * B_PER_W

        def idx_copy(i, b):
            return pltpu.make_async_copy(
                idx_hbm.at[pl.ds(base + i * CHUNK, CHUNK)], idx_v[b], isem[b])

        def gather_copy(b):
            return pltpu.make_async_copy(
                table_hbm.at[idx_v[b]], rows_v[b], gsem[b])

        def store_copy(j, b):
            return pltpu.make_async_copy(
                rows_v[b], out_hbm.at[pl.ds(base + j * CHUNK, CHUNK)],
                ssem[b])

        for b in range(DEPTH):
            idx_copy(b, b).start()

        for i in range(NCHUNK):
            b = i % DEPTH
            idx_copy(i, b).wait()
            if i >= DEPTH:
                store_copy(i - DEPTH, b).wait()
            gather_copy(b).start()
            j = i - (DEPTH - 1)
            if j >= 0:
                bj = j % DEPTH
                gather_copy(bj).wait()
                store_copy(j, bj).start()
                if j + DEPTH < NCHUNK:
                    idx_copy(j + DEPTH, bj).start()

        for j in range(NCHUNK - (DEPTH - 1), NCHUNK):
            bj = j % DEPTH
            gather_copy(bj).wait()
            store_copy(j, bj).start()
        for j in range(NCHUNK - DEPTH, NCHUNK):
            store_copy(j, j % DEPTH).wait()

    return k(table, idx_flat)


@jax.jit
def kernel(inputs, embedding):
    idx_flat = jnp.reshape(inputs, (B,)).astype(jnp.int32)
    out = _sc_gather(embedding, idx_flat)
    return jnp.reshape(out, (BATCH, HIST, EMBED))
